# baseline (speedup 1.0000x reference)
import functools

import jax
import jax.numpy as jnp
from jax import lax
from jax.experimental import pallas as pl
from jax.experimental.pallas import tpu as pltpu

_K = 16
_STEPS = 13
_HID = 128


def _tc_body(pidx, pq, *refs):
    wblks = refs[0:_K]
    spk_blk = refs[_K]
    w1_blk = refs[_K + 1]
    w1_spk = refs[_K + 2]
    w1_qnt = refs[_K + 3]
    b1_ref = refs[_K + 4]
    w2_ref = refs[_K + 5]
    out_ref = refs[_K + 6]
    h_ref = refs[_K + 7]
    i = pl.program_id(0)

    lane = lax.broadcasted_iota(jnp.int32, (1, 128), 1)

    def extract(blk, col):
        oh = (lane == col).astype(jnp.float32)
        return lax.dot_general(oh, blk[...], (((1,), (1,)), ((), ())),
                               preferred_element_type=jnp.float32)

    @pl.when(i == 0)
    def _():
        espk = extract(spk_blk, pidx[0] % 128)
        h_ref[...] = lax.dot_general(
            espk, w1_spk[...], (((1,), (0,)), ((), ())),
            preferred_element_type=jnp.float32)

    embs = [extract(wblks[k], pidx[1 + i * _K + k] % 128) for k in range(_K)]
    e_row = jnp.concatenate(embs, axis=1)
    h_ref[...] += lax.dot_general(
        e_row, w1_blk[...], (((1,), (0,)), ((), ())),
        preferred_element_type=jnp.float32)

    @pl.when(i == _STEPS - 1)
    def _():
        h = h_ref[...] + pq[0] * w1_qnt[0:1, :] + b1_ref[...]
        h = jnp.maximum(h, 0.0)
        s = jnp.sum(h * w2_ref[...])
        out_ref[...] = jnp.full((1, 1), 1.0 / (1.0 + jnp.exp(-(s + pq[1]))))


def kernel(context_indices, speaker, col_three_indices, quant, sentiment,
           word_emb, speaker_emb, W1, b1, W2, b2):
    del sentiment
    ctx = context_indices.astype(jnp.int32)
    c3 = col_three_indices.astype(jnp.int32)
    pidx = jnp.concatenate(
        [speaker.astype(jnp.int32), ctx, c3, jnp.broadcast_to(c3, (8,))]
    )
    pq = jnp.concatenate([quant.astype(jnp.float32), b2.astype(jnp.float32)])

    wordT = word_emb.T
    spkT = speaker_emb.T
    w1T = W1.T
    w1s = jnp.concatenate([
        w1T[16:3216],
        jnp.zeros((128, _HID), jnp.float32),
        w1T[0:16],
        w1T[3216:3217],
        jnp.zeros((15, _HID), jnp.float32),
    ])
    b1r = b1.reshape(1, _HID)

    word_specs = [
        pl.BlockSpec(
            (16, 128),
            functools.partial(
                lambda kk, i, pidx, pq: (0, pidx[1 + i * _K + kk] // 128), k
            ),
        )
        for k in range(_K)
    ]
    spk_spec = pl.BlockSpec((16, 128), lambda i, pidx, pq: (0, pidx[0] // 128))
    w1blk_spec = pl.BlockSpec((16 * _K, 128), lambda i, pidx, pq: (i, 0))
    w1spk_spec = pl.BlockSpec((16, 128), lambda i, pidx, pq: (208, 0))
    w1qnt_spec = pl.BlockSpec((16, 128), lambda i, pidx, pq: (209, 0))
    b1_spec = pl.BlockSpec((1, _HID), lambda i, pidx, pq: (0, 0))
    w2_spec = pl.BlockSpec((1, _HID), lambda i, pidx, pq: (0, 0))
    out_spec = pl.BlockSpec((1, 1), lambda i, pidx, pq: (0, 0))

    grid_spec = pltpu.PrefetchScalarGridSpec(
        num_scalar_prefetch=2,
        grid=(_STEPS,),
        in_specs=word_specs
        + [spk_spec, w1blk_spec, w1spk_spec, w1qnt_spec, b1_spec, w2_spec],
        out_specs=out_spec,
        scratch_shapes=[pltpu.VMEM((1, _HID), jnp.float32)],
    )
    out = pl.pallas_call(
        _tc_body,
        grid_spec=grid_spec,
        out_shape=jax.ShapeDtypeStruct((1, 1), jnp.float32),
    )(
        pidx, pq,
        *([wordT] * _K), spkT, w1s, w1s, w1s, b1r, W2,
    )
    return out


# speedup vs baseline: 1.0394x; 1.0394x over previous
import functools

import jax
import jax.numpy as jnp
from jax import lax
from jax.experimental import pallas as pl
from jax.experimental.pallas import tpu as pltpu

_K = 16
_STEPS = 13
_HID = 128


def _tc_body(pidx, pq, *refs):
    wblks = refs[0:_K]
    spk_blk = refs[_K]
    w1_blk = refs[_K + 1]
    w1_spk = refs[_K + 2]
    w1_qnt = refs[_K + 3]
    b1_ref = refs[_K + 4]
    w2_ref = refs[_K + 5]
    out_ref = refs[_K + 6]
    h_ref = refs[_K + 7]
    i = pl.program_id(0)

    lane = lax.broadcasted_iota(jnp.int32, (1, 128), 1)

    def extract(blk, col):
        oh = (lane == col).astype(jnp.float32)
        return lax.dot_general(oh, blk[...], (((1,), (1,)), ((), ())),
                               preferred_element_type=jnp.float32)

    @pl.when(i == 0)
    def _():
        espk = extract(spk_blk, pidx[1, 0])
        h_ref[...] = lax.dot_general(
            espk, w1_spk[...], (((1,), (0,)), ((), ())),
            preferred_element_type=jnp.float32)

    embs = [extract(wblks[k], pidx[1, 1 + i * _K + k]) for k in range(_K)]
    e_row = jnp.concatenate(embs, axis=1)
    h_ref[...] += lax.dot_general(
        e_row, w1_blk[...], (((1,), (0,)), ((), ())),
        preferred_element_type=jnp.float32)

    @pl.when(i == _STEPS - 1)
    def _():
        h = h_ref[...] + pq[0] * w1_qnt[0:1, :] + b1_ref[...]
        h = jnp.maximum(h, 0.0)
        s = jnp.sum(h * w2_ref[...])
        out_ref[...] = jnp.full((1, 1), 1.0 / (1.0 + jnp.exp(-(s + pq[1]))))


def kernel(context_indices, speaker, col_three_indices, quant, sentiment,
           word_emb, speaker_emb, W1, b1, W2, b2):
    del sentiment
    ctx = context_indices.astype(jnp.int32)
    c3 = col_three_indices.astype(jnp.int32)
    widx = jnp.concatenate(
        [speaker.astype(jnp.int32), ctx, c3, jnp.broadcast_to(c3, (8,))]
    )
    pidx = jnp.stack([widx // 128, widx % 128])
    pq = jnp.concatenate([quant.astype(jnp.float32), b2.astype(jnp.float32)])

    wordT = word_emb.T
    spkT = speaker_emb.T
    w1T = W1.T
    w1s = jnp.concatenate([
        w1T[16:3216],
        jnp.zeros((128, _HID), jnp.float32),
        w1T[0:16],
        w1T[3216:3217],
        jnp.zeros((15, _HID), jnp.float32),
    ])
    b1r = b1.reshape(1, _HID)

    word_specs = [
        pl.BlockSpec(
            (16, 128),
            functools.partial(
                lambda kk, i, pidx, pq: (0, pidx[0, 1 + i * _K + kk]), k
            ),
        )
        for k in range(_K)
    ]
    spk_spec = pl.BlockSpec((16, 128), lambda i, pidx, pq: (0, pidx[0, 0]))
    w1blk_spec = pl.BlockSpec((16 * _K, 128), lambda i, pidx, pq: (i, 0))
    w1spk_spec = pl.BlockSpec((16, 128), lambda i, pidx, pq: (208, 0))
    w1qnt_spec = pl.BlockSpec((16, 128), lambda i, pidx, pq: (209, 0))
    b1_spec = pl.BlockSpec((1, _HID), lambda i, pidx, pq: (0, 0))
    w2_spec = pl.BlockSpec((1, _HID), lambda i, pidx, pq: (0, 0))
    out_spec = pl.BlockSpec((1, 1), lambda i, pidx, pq: (0, 0))

    grid_spec = pltpu.PrefetchScalarGridSpec(
        num_scalar_prefetch=2,
        grid=(_STEPS,),
        in_specs=word_specs
        + [spk_spec, w1blk_spec, w1spk_spec, w1qnt_spec, b1_spec, w2_spec],
        out_specs=out_spec,
        scratch_shapes=[pltpu.VMEM((1, _HID), jnp.float32)],
    )
    out = pl.pallas_call(
        _tc_body,
        grid_spec=grid_spec,
        out_shape=jax.ShapeDtypeStruct((1, 1), jnp.float32),
    )(
        pidx, pq,
        *([wordT] * _K), spkT, w1s, w1s, w1s, b1r, W2,
    )
    return out
